# baseline (speedup 1.0000x reference)
import functools

import jax
import jax.numpy as jnp
from jax import lax
from jax.experimental import pallas as pl
from jax.experimental.pallas import tpu as pltpu
from jax.experimental.pallas import tpu_sc as plsc

N_NODES = 10000
N_EDGES = 160000
D = 256
NC = 2
NS = 16
NW = NC * NS
E_PAD = 163840
EPT_A = E_PAD // NW
DEG_PAD = 10240
WS = 312
WS_LAST = N_NODES - (NW - 1) * WS
ACC_ROWS = 336
SINK = 328
B = 128
CHUNK = 1024

_sc_mesh = plsc.VectorSubcoreMesh(core_axis_name="c", subcore_axis_name="s")
_sc_params = pltpu.CompilerParams(needs_layout_passes=False)


@functools.partial(
    pl.kernel,
    out_type=jax.ShapeDtypeStruct((NC * DEG_PAD,), jnp.float32),
    mesh=_sc_mesh,
    compiler_params=_sc_params,
    scratch_types=[
        pltpu.VMEM((EPT_A,), jnp.int32),
        pltpu.VMEM((B,), jnp.int32),
        pltpu.VMEM((B,), jnp.float32),
        pltpu.VMEM((640,), jnp.float32),
        pltpu.VMEM_SHARED((DEG_PAD,), jnp.float32),
    ],
)
def _deg_kernel(src, degp, srcb, idxb, ones, zb, dacc):
    c = lax.axis_index("c")
    s = lax.axis_index("s")
    wid = s * NC + c

    def fill_ones(i, _):
        ones[pl.ds(i * 16, 16)] = jnp.ones((16,), jnp.float32)
        return 0

    lax.fori_loop(0, B // 16, fill_ones, 0)

    def fill_z(i, _):
        zb[pl.ds(i * 16, 16)] = jnp.zeros((16,), jnp.float32)
        return 0

    lax.fori_loop(0, 640 // 16, fill_z, 0)

    pltpu.sync_copy(zb, dacc.at[pl.ds(s * 640, 640)])
    pltpu.sync_copy(src.at[pl.ds(wid * EPT_A, EPT_A)], srcb)
    plsc.subcore_barrier()

    def batch(b, _):
        off = pl.multiple_of(b * B, B)
        for k in range(B // 16):
            idxb[pl.ds(k * 16, 16)] = srcb[pl.ds(off + k * 16, 16)]
        pltpu.sync_copy(ones, dacc.at[idxb], add=True)
        return 0

    lax.fori_loop(0, EPT_A // B, batch, 0)
    plsc.subcore_barrier()
    pltpu.sync_copy(dacc.at[pl.ds(s * 640, 640)],
                    degp.at[pl.ds(c * DEG_PAD + s * 640, 640)])


ROWS_BLK = 2000


def _mm_body(deg_ref, feat_ref, w_ref, out_ref):
    h = jnp.dot(feat_ref[...], w_ref[...], preferred_element_type=jnp.float32)
    deg = deg_ref[:, 0] + deg_ref[:, 1]
    norm = lax.rsqrt(jnp.maximum(deg, 1.0))
    out_ref[...] = h * norm[:, None]


_mm = pl.pallas_call(
    _mm_body,
    grid=(N_NODES // ROWS_BLK,),
    in_specs=[
        pl.BlockSpec((ROWS_BLK, NC), lambda i: (i, 0)),
        pl.BlockSpec((ROWS_BLK, D), lambda i: (i, 0)),
        pl.BlockSpec((D, D), lambda i: (0, 0)),
    ],
    out_specs=pl.BlockSpec((ROWS_BLK, D), lambda i: (i, 0)),
    out_shape=jax.ShapeDtypeStruct((N_NODES, D), jnp.float32),
)


@functools.partial(
    pl.kernel,
    out_type=jax.ShapeDtypeStruct((N_NODES, D), jnp.float32),
    mesh=_sc_mesh,
    compiler_params=_sc_params,
    scratch_types=[
        pltpu.VMEM((CHUNK,), jnp.int32),
        pltpu.VMEM((CHUNK,), jnp.int32),
        pltpu.VMEM((CHUNK + B,), jnp.int32),
        pltpu.VMEM((CHUNK + B,), jnp.int32),
        pltpu.VMEM((B, D), jnp.float32),
        pltpu.VMEM((ACC_ROWS, D), jnp.float32),
        pltpu.SemaphoreType.DMA,
    ],
)
def _agg_kernel(hn, src, dst, out, srcb, dstb, sel_src, sel_loc, rows, acc,
                sem):
    c = lax.axis_index("c")
    s = lax.axis_index("s")
    w = s * NC + c
    lo = w * WS
    hi = jnp.where(w == NW - 1, N_NODES, lo + WS)

    def zrow(r, _):
        for k in range(D // 16):
            acc[r, pl.ds(k * 16, 16)] = jnp.zeros((16,), jnp.float32)
        return 0

    lax.fori_loop(0, ACC_ROWS, zrow, 0)

    zed = jnp.zeros((16,), jnp.int32)
    snk = jnp.full((16,), SINK, jnp.int32)

    def chunk_body(ci, _):
        coff = pl.multiple_of(ci * CHUNK, CHUNK)
        pltpu.sync_copy(src.at[pl.ds(coff, CHUNK)], srcb)
        pltpu.sync_copy(dst.at[pl.ds(coff, CHUNK)], dstb)

        def cvreg(i, wp):
            off = pl.multiple_of(i * 16, 16)
            sv = srcb[pl.ds(off, 16)]
            dv = dstb[pl.ds(off, 16)]
            m = (dv >= lo) & (dv < hi)
            pc = plsc.all_reduce_population_count(m)
            plsc.store_compressed(sel_src.at[pl.ds(wp, 16)], sv, mask=m)
            plsc.store_compressed(sel_loc.at[pl.ds(wp, 16)], dv - lo, mask=m)
            return wp + pc[0]

        wp = lax.fori_loop(0, CHUNK // 16, cvreg, 0)

        for j in range(B // 16):
            sel_src[pl.ds(wp + j * 16, 16)] = zed
            sel_loc[pl.ds(wp + j * 16, 16)] = snk

        nb = (wp + B - 1) // B

        def batch_body(b, _):
            off = pl.multiple_of(b * B, B)
            pltpu.async_copy(hn.at[sel_src.at[pl.ds(off, B)]], rows,
                             sem).wait()

            def add_group(j, _):
                goff = pl.multiple_of(j * 16, 16)
                lvec = sel_loc[pl.ds(off + goff, 16)]
                for jj in range(16):
                    loc = lvec[jj]
                    for k in range(D // 16):
                        plsc.addupdate(acc.at[loc, pl.ds(k * 16, 16)],
                                       rows[goff + jj, pl.ds(k * 16, 16)])
                return 0

            lax.fori_loop(0, B // 16, add_group, 0)
            return 0

        lax.fori_loop(0, nb, batch_body, 0)
        return 0

    lax.fori_loop(0, E_PAD // CHUNK, chunk_body, 0)

    def relu_rows(nrows):
        def rbody(r, _):
            for k in range(D // 16):
                v = acc[r, pl.ds(k * 16, 16)]
                acc[r, pl.ds(k * 16, 16)] = jnp.maximum(v, 0.0)
            return 0

        lax.fori_loop(0, nrows, rbody, 0)

    @pl.when(w < NW - 1)
    def _():
        relu_rows(WS)
        pltpu.sync_copy(acc.at[pl.ds(0, WS)], out.at[pl.ds(lo, WS)])

    @pl.when(w == NW - 1)
    def _():
        relu_rows(WS_LAST)
        pltpu.sync_copy(acc.at[pl.ds(0, WS_LAST)], out.at[pl.ds(lo, WS_LAST)])


def kernel(feature, edge_index, weight):
    npad = E_PAD - N_EDGES
    src = jnp.concatenate([edge_index[0], jnp.full((npad,), N_NODES, jnp.int32)])
    dst = jnp.concatenate(
        [edge_index[1], jnp.full((npad,), 2 * N_NODES, jnp.int32)])
    degp = _deg_kernel(src)
    hn = _mm(degp.reshape(NC, DEG_PAD).T, feature, weight)
    return _agg_kernel(hn, src, dst)


# speedup vs baseline: 2.8984x; 2.8984x over previous
import functools

import jax
import jax.numpy as jnp
from jax import lax
from jax.experimental import pallas as pl
from jax.experimental.pallas import tpu as pltpu
from jax.experimental.pallas import tpu_sc as plsc

N_NODES = 10000
N_EDGES = 160000
D = 256
NC = 2
NS = 16
NW = NC * NS
E_PAD = 163840
EPT_A = E_PAD // NW
DEG_PAD = 10240
WS = 312
WS_LAST = N_NODES - (NW - 1) * WS
ACC_ROWS = 336
SINK = 328
B = 128
CHUNK = 2048

_sc_mesh = plsc.VectorSubcoreMesh(core_axis_name="c", subcore_axis_name="s")
_sc_params = pltpu.CompilerParams(needs_layout_passes=False)


@functools.partial(
    pl.kernel,
    out_type=jax.ShapeDtypeStruct((NC * DEG_PAD,), jnp.float32),
    mesh=_sc_mesh,
    compiler_params=_sc_params,
    scratch_types=[
        pltpu.VMEM((EPT_A,), jnp.int32),
        pltpu.VMEM((B,), jnp.int32),
        pltpu.VMEM((B,), jnp.float32),
        pltpu.VMEM((640,), jnp.float32),
        pltpu.VMEM_SHARED((DEG_PAD,), jnp.float32),
    ],
)
def _deg_kernel(src, degp, srcb, idxb, ones, zb, dacc):
    c = lax.axis_index("c")
    s = lax.axis_index("s")
    wid = s * NC + c

    def fill_ones(i, _):
        ones[pl.ds(i * 16, 16)] = jnp.ones((16,), jnp.float32)
        return 0

    lax.fori_loop(0, B // 16, fill_ones, 0)

    def fill_z(i, _):
        zb[pl.ds(i * 16, 16)] = jnp.zeros((16,), jnp.float32)
        return 0

    lax.fori_loop(0, 640 // 16, fill_z, 0)

    pltpu.sync_copy(zb, dacc.at[pl.ds(s * 640, 640)])
    pltpu.sync_copy(src.at[pl.ds(wid * EPT_A, EPT_A)], srcb)
    plsc.subcore_barrier()

    def batch(b, _):
        off = pl.multiple_of(b * B, B)
        for k in range(B // 16):
            idxb[pl.ds(k * 16, 16)] = srcb[pl.ds(off + k * 16, 16)]
        pltpu.sync_copy(ones, dacc.at[idxb], add=True)
        return 0

    lax.fori_loop(0, EPT_A // B, batch, 0)
    plsc.subcore_barrier()
    pltpu.sync_copy(dacc.at[pl.ds(s * 640, 640)],
                    degp.at[pl.ds(c * DEG_PAD + s * 640, 640)])


ROWS_BLK = 2000


def _mm_body(deg_ref, feat_ref, w_ref, out_ref):
    h = jnp.dot(feat_ref[...], w_ref[...], preferred_element_type=jnp.float32)
    deg = deg_ref[:, 0] + deg_ref[:, 1]
    norm = lax.rsqrt(jnp.maximum(deg, 1.0))
    out_ref[...] = h * norm[:, None]


_mm = pl.pallas_call(
    _mm_body,
    grid=(N_NODES // ROWS_BLK,),
    in_specs=[
        pl.BlockSpec((ROWS_BLK, NC), lambda i: (i, 0)),
        pl.BlockSpec((ROWS_BLK, D), lambda i: (i, 0)),
        pl.BlockSpec((D, D), lambda i: (0, 0)),
    ],
    out_specs=pl.BlockSpec((ROWS_BLK, D), lambda i: (i, 0)),
    out_shape=jax.ShapeDtypeStruct((N_NODES, D), jnp.float32),
)


@functools.partial(
    pl.kernel,
    out_type=jax.ShapeDtypeStruct((N_NODES, D), jnp.float32),
    mesh=_sc_mesh,
    compiler_params=_sc_params,
    scratch_types=[
        pltpu.VMEM((CHUNK,), jnp.int32),
        pltpu.VMEM((CHUNK,), jnp.int32),
        pltpu.VMEM((CHUNK + B,), jnp.int32),
        pltpu.VMEM((CHUNK + B,), jnp.int32),
        pltpu.VMEM((B, D), jnp.float32),
        pltpu.VMEM((ACC_ROWS, D), jnp.float32),
        pltpu.SemaphoreType.DMA,
    ],
)
def _agg_kernel(hn, src, dst, out, srcb, dstb, sel_src, sel_loc, rows, acc,
                sem):
    c = lax.axis_index("c")
    s = lax.axis_index("s")
    w = s * NC + c
    lo = w * WS
    hi = jnp.where(w == NW - 1, N_NODES, lo + WS)

    def zrow(r, _):
        for k in range(D // 16):
            acc[r, pl.ds(k * 16, 16)] = jnp.zeros((16,), jnp.float32)
        return 0

    lax.fori_loop(0, ACC_ROWS, zrow, 0)

    zed = jnp.zeros((16,), jnp.int32)
    snk = jnp.full((16,), SINK, jnp.int32)

    def chunk_body(ci, _):
        coff = pl.multiple_of(ci * CHUNK, CHUNK)
        pltpu.sync_copy(src.at[pl.ds(coff, CHUNK)], srcb)
        pltpu.sync_copy(dst.at[pl.ds(coff, CHUNK)], dstb)

        def cvreg(i, wp):
            off = pl.multiple_of(i * 16, 16)
            sv = srcb[pl.ds(off, 16)]
            dv = dstb[pl.ds(off, 16)]
            m = (dv >= lo) & (dv < hi)
            pc = plsc.all_reduce_population_count(m)
            plsc.store_compressed(sel_src.at[pl.ds(wp, 16)], sv, mask=m)
            plsc.store_compressed(sel_loc.at[pl.ds(wp, 16)], dv - lo, mask=m)
            return wp + pc[0]

        wp = lax.fori_loop(0, CHUNK // 16, cvreg, 0)

        for j in range(B // 16):
            sel_src[pl.ds(wp + j * 16, 16)] = zed
            sel_loc[pl.ds(wp + j * 16, 16)] = snk

        nb = (wp + B - 1) // B

        def batch_body(b, _):
            off = pl.multiple_of(b * B, B)
            pltpu.async_copy(hn.at[sel_src.at[pl.ds(off, B)]], rows,
                             sem).wait()

            def add_group(j, _):
                goff = pl.multiple_of(j * 16, 16)
                lvec = sel_loc[pl.ds(off + goff, 16)]
                for jj in range(16):
                    loc = lvec[jj]
                    for k in range(D // 16):
                        plsc.addupdate(acc.at[loc, pl.ds(k * 16, 16)],
                                       rows[goff + jj, pl.ds(k * 16, 16)])
                return 0

            lax.fori_loop(0, B // 16, add_group, 0)
            return 0

        lax.fori_loop(0, nb, batch_body, 0)
        return 0

    lax.fori_loop(0, E_PAD // CHUNK, chunk_body, 0)

    def relu_rows(nrows):
        def rbody(r, _):
            for k in range(D // 16):
                v = acc[r, pl.ds(k * 16, 16)]
                acc[r, pl.ds(k * 16, 16)] = jnp.maximum(v, 0.0)
            return 0

        lax.fori_loop(0, nrows, rbody, 0)

    @pl.when(w < NW - 1)
    def _():
        relu_rows(WS)
        pltpu.sync_copy(acc.at[pl.ds(0, WS)], out.at[pl.ds(lo, WS)])

    @pl.when(w == NW - 1)
    def _():
        relu_rows(WS_LAST)
        pltpu.sync_copy(acc.at[pl.ds(0, WS_LAST)], out.at[pl.ds(lo, WS_LAST)])


def kernel(feature, edge_index, weight):
    npad = E_PAD - N_EDGES
    src = jnp.concatenate([edge_index[0], jnp.full((npad,), N_NODES, jnp.int32)])
    dst = jnp.concatenate(
        [edge_index[1], jnp.full((npad,), 2 * N_NODES, jnp.int32)])
    degp = _deg_kernel(src)
    hn = _mm(degp.reshape(NC, DEG_PAD).T, feature, weight)
    return _agg_kernel(hn, src, dst)
